# baseline (speedup 1.0000x reference)
import jax
import jax.numpy as jnp
from jax import lax
from jax.experimental import pallas as pl
from jax.experimental.pallas import tpu as pltpu
from jax.experimental.pallas import tpu_sc as plsc

BATCH = 16384
N_FEATURES = 26
DIM = 32

NC, NS = 2, 16
NW = NC * NS
B_PER_W = BATCH // NW
BB_PER_W = B_PER_W // 128
NIT = BB_PER_W * N_FEATURES


def _lookup_kernel(table_hbm, idx_hbm, out_hbm, idx_v, rows, tiles, isem, gsem, ssem):
    wid = lax.axis_index("s") * NC + lax.axis_index("c")
    iota = lax.iota(jnp.int32, 16)

    pltpu.async_copy(
        idx_hbm.at[:, pl.ds(wid * BB_PER_W, BB_PER_W)], idx_v, isem
    ).wait()

    def start_gathers(f, slot):
        for bbl in range(BB_PER_W):
            pltpu.async_copy(
                table_hbm.at[idx_v.at[f, bbl]],
                rows.at[slot, pl.ds(bbl * 128, 128)],
                gsem.at[slot],
            )

    def wait_gathers(slot):
        pltpu.make_async_copy(
            table_hbm.at[pl.ds(0, 4 * 128)], rows.at[slot], gsem.at[slot]
        ).wait()

    def start_stores(f, slot):
        for d8 in range(4):
            pltpu.async_copy(
                tiles.at[slot, d8],
                out_hbm.at[f, d8, pl.ds(wid * BB_PER_W, BB_PER_W)],
                ssem.at[slot],
            )

    def wait_stores(slot):
        for _ in range(4):
            pltpu.make_async_copy(
                tiles.at[slot, 0], out_hbm.at[0, 0, pl.ds(0, BB_PER_W)],
                ssem.at[slot],
            ).wait()

    def transpose_block(slot):
        for d8 in range(4):
            for dr in range(8):
                col = jnp.full((16,), d8 * 8 + dr, jnp.int32)
                for bbl in range(BB_PER_W):
                    for b16 in range(8):
                        vals = plsc.load_gather(
                            rows.at[slot],
                            [bbl * 128 + b16 * 16 + iota, col],
                        )
                        tiles[slot, d8, bbl, dr, pl.ds(b16 * 16, 16)] = vals

    start_gathers(0, 0)

    def body(f, carry):
        slot = f & 1
        other = 1 - slot

        @pl.when(f + 1 < N_FEATURES)
        def _():
            start_gathers(f + 1, other)

        wait_gathers(slot)

        @pl.when(f >= 2)
        def _():
            wait_stores(slot)

        transpose_block(slot)
        start_stores(f, slot)
        return carry

    lax.fori_loop(0, N_FEATURES, body, 0)
    wait_stores(0)
    wait_stores(1)


def kernel(indices, table):
    idx3 = indices.T.reshape(N_FEATURES, BATCH // 128, 128)
    mesh = plsc.VectorSubcoreMesh(core_axis_name="c", subcore_axis_name="s")
    out5 = pl.kernel(
        _lookup_kernel,
        out_type=jax.ShapeDtypeStruct(
            (N_FEATURES, 4, BATCH // 128, 8, 128), jnp.float32
        ),
        mesh=mesh,
        scratch_types=[
            pltpu.VMEM((N_FEATURES, BB_PER_W, 128), jnp.int32),
            pltpu.VMEM((2, BB_PER_W * 128, DIM), jnp.float32),
            pltpu.VMEM((2, 4, BB_PER_W, 8, 128), jnp.float32),
            pltpu.SemaphoreType.DMA,
            pltpu.SemaphoreType.DMA((2,)),
            pltpu.SemaphoreType.DMA((2,)),
        ],
        compiler_params=pltpu.CompilerParams(
            use_tc_tiling_on_sc=False, needs_layout_passes=False
        ),
    )(table, idx3)
    return out5.transpose(2, 4, 0, 1, 3).reshape(BATCH, N_FEATURES, DIM)


# speedup vs baseline: 1.0076x; 1.0076x over previous
import jax
import jax.numpy as jnp
from jax import lax
from jax.experimental import pallas as pl
from jax.experimental.pallas import tpu as pltpu
from jax.experimental.pallas import tpu_sc as plsc

BATCH = 16384
N_FEATURES = 26
DIM = 32

NC, NS = 2, 16
NW = NC * NS
B_PER_W = BATCH // NW
BB_PER_W = B_PER_W // 128
NIT = BB_PER_W * N_FEATURES


def _lookup_kernel(table_hbm, idx_hbm, out_hbm, idx_v, rows, tiles, isem, gsem, ssem):
    wid = lax.axis_index("s") * NC + lax.axis_index("c")
    iota = lax.iota(jnp.int32, 16)

    pltpu.async_copy(
        idx_hbm.at[:, pl.ds(wid * BB_PER_W, BB_PER_W)], idx_v, isem
    ).wait()

    def start_gathers(f, slot):
        for bbl in range(BB_PER_W):
            pltpu.async_copy(
                table_hbm.at[idx_v.at[f, bbl]],
                rows.at[slot, pl.ds(bbl * 128, 128)],
                gsem.at[slot],
            )

    def wait_gathers(slot):
        pltpu.make_async_copy(
            table_hbm.at[pl.ds(0, 4 * 128)], rows.at[slot], gsem.at[slot]
        ).wait()

    def start_stores(f, slot):
        for d8 in range(4):
            pltpu.async_copy(
                tiles.at[slot, d8],
                out_hbm.at[f, d8, pl.ds(wid * BB_PER_W, BB_PER_W)],
                ssem.at[slot],
            )

    def wait_stores(slot):
        for _ in range(4):
            pltpu.make_async_copy(
                tiles.at[slot, 0], out_hbm.at[0, 0, pl.ds(0, BB_PER_W)],
                ssem.at[slot],
            ).wait()

    def transpose_block(slot):
        def dbody(d, carry):
            col = jnp.broadcast_to(d, (16,)).astype(jnp.int32)
            d8, dr = d // 8, lax.rem(d, 8)
            for bbl in range(BB_PER_W):
                for b16 in range(8):
                    vals = plsc.load_gather(
                        rows.at[slot],
                        [bbl * 128 + b16 * 16 + iota, col],
                    )
                    tiles[slot, d8, bbl, dr, pl.ds(b16 * 16, 16)] = vals
            return carry

        lax.fori_loop(0, DIM, dbody, 0)

    start_gathers(0, 0)

    def body(f, carry):
        slot = f & 1
        other = 1 - slot

        @pl.when(f + 1 < N_FEATURES)
        def _():
            start_gathers(f + 1, other)

        wait_gathers(slot)

        @pl.when(f >= 2)
        def _():
            wait_stores(slot)

        transpose_block(slot)
        start_stores(f, slot)
        return carry

    lax.fori_loop(0, N_FEATURES, body, 0)
    wait_stores(0)
    wait_stores(1)


def kernel(indices, table):
    idx3 = indices.T.reshape(N_FEATURES, BATCH // 128, 128)
    mesh = plsc.VectorSubcoreMesh(core_axis_name="c", subcore_axis_name="s")
    out5 = pl.kernel(
        _lookup_kernel,
        out_type=jax.ShapeDtypeStruct(
            (N_FEATURES, 4, BATCH // 128, 8, 128), jnp.float32
        ),
        mesh=mesh,
        scratch_types=[
            pltpu.VMEM((N_FEATURES, BB_PER_W, 128), jnp.int32),
            pltpu.VMEM((2, BB_PER_W * 128, DIM), jnp.float32),
            pltpu.VMEM((2, 4, BB_PER_W, 8, 128), jnp.float32),
            pltpu.SemaphoreType.DMA,
            pltpu.SemaphoreType.DMA((2,)),
            pltpu.SemaphoreType.DMA((2,)),
        ],
        compiler_params=pltpu.CompilerParams(
            use_tc_tiling_on_sc=False, needs_layout_passes=False
        ),
    )(table, idx3)
    return out5.transpose(2, 4, 0, 1, 3).reshape(BATCH, N_FEATURES, DIM)


# speedup vs baseline: 1.3499x; 1.3398x over previous
import jax
import jax.numpy as jnp
from jax import lax
from jax.experimental import pallas as pl
from jax.experimental.pallas import tpu as pltpu
from jax.experimental.pallas import tpu_sc as plsc

BATCH = 16384
N_FEATURES = 26
DIM = 32

NC, NS = 2, 16
NW = NC * NS
B_PER_W = BATCH // NW
BB_PER_W = B_PER_W // 128
NIT = BB_PER_W * N_FEATURES


def _lookup_kernel(table_hbm, idx_hbm, out_hbm, idx_v, rows, tiles, isem, gsem, ssem):
    wid = lax.axis_index("s") * NC + lax.axis_index("c")
    iota = lax.iota(jnp.int32, 16)

    pltpu.async_copy(
        idx_hbm.at[:, pl.ds(wid * BB_PER_W, BB_PER_W)], idx_v, isem
    ).wait()

    def start_gathers(f, slot):
        for bbl in range(BB_PER_W):
            pltpu.async_copy(
                table_hbm.at[idx_v.at[f, bbl]],
                rows.at[slot, pl.ds(bbl * 128, 128)],
                gsem.at[slot],
            )

    def wait_gathers(slot):
        pltpu.make_async_copy(
            table_hbm.at[pl.ds(0, 4 * 128)], rows.at[slot], gsem.at[slot]
        ).wait()

    def start_stores(f, slot):
        for d8 in range(4):
            pltpu.async_copy(
                tiles.at[slot, d8, pl.ds(0, BB_PER_W), :, pl.ds(0, 128)],
                out_hbm.at[f, d8, pl.ds(wid * BB_PER_W, BB_PER_W)],
                ssem.at[slot],
            )

    def wait_stores(slot):
        for _ in range(4):
            pltpu.make_async_copy(
                tiles.at[slot, 0, pl.ds(0, BB_PER_W), :, pl.ds(0, 128)],
                out_hbm.at[0, 0, pl.ds(0, BB_PER_W)],
                ssem.at[slot],
            ).wait()

    d8_vec = [iota // 8, 2 + iota // 8]
    dr_vec = iota & 7

    def transpose_block(slot):
        def rbody(r0, carry):
            for u in range(8):
                r = r0 * 8 + u
                bbl = r // 128
                br = jnp.broadcast_to(lax.rem(r, 128), (16,)).astype(jnp.int32)
                bblv = jnp.broadcast_to(bbl, (16,)).astype(jnp.int32)
                for h in range(2):
                    vals = rows[slot, r, pl.ds(h * 16, 16)]
                    plsc.store_scatter(
                        tiles.at[slot], [d8_vec[h], bblv, dr_vec, br], vals
                    )
            return carry

        lax.fori_loop(0, 4 * 128 // 8, rbody, 0)

    start_gathers(0, 0)

    def body(f, carry):
        slot = f & 1
        other = 1 - slot

        @pl.when(f + 1 < N_FEATURES)
        def _():
            start_gathers(f + 1, other)

        wait_gathers(slot)

        @pl.when(f >= 2)
        def _():
            wait_stores(slot)

        transpose_block(slot)
        start_stores(f, slot)
        return carry

    lax.fori_loop(0, N_FEATURES, body, 0)
    wait_stores(0)
    wait_stores(1)


def kernel(indices, table):
    idx3 = indices.T.reshape(N_FEATURES, BATCH // 128, 128)
    mesh = plsc.VectorSubcoreMesh(core_axis_name="c", subcore_axis_name="s")
    out5 = pl.kernel(
        _lookup_kernel,
        out_type=jax.ShapeDtypeStruct(
            (N_FEATURES, 4, BATCH // 128, 8, 128), jnp.float32
        ),
        mesh=mesh,
        scratch_types=[
            pltpu.VMEM((N_FEATURES, BB_PER_W, 128), jnp.int32),
            pltpu.VMEM((2, BB_PER_W * 128, DIM), jnp.float32),
            pltpu.VMEM((2, 4, BB_PER_W + 1, 8, 129), jnp.float32),
            pltpu.SemaphoreType.DMA,
            pltpu.SemaphoreType.DMA((2,)),
            pltpu.SemaphoreType.DMA((2,)),
        ],
        compiler_params=pltpu.CompilerParams(
            use_tc_tiling_on_sc=False, needs_layout_passes=False
        ),
    )(table, idx3)
    return out5.transpose(2, 4, 0, 1, 3).reshape(BATCH, N_FEATURES, DIM)
